# baseline (speedup 1.0000x reference)
import functools

import jax
import jax.numpy as jnp
from jax import lax
from jax.experimental import pallas as pl
from jax.experimental.pallas import tpu as pltpu
from jax.experimental.pallas import tpu_sc as plsc

EMBED = 64
HIDDEN = 256
NC = 2
NS = 16
NW = NC * NS


def _sc_gather(user_pairs, item_pairs, user_hids, item_hids):
    B = user_hids.shape[0]
    assert B % (8 * NW) == 0
    b_per_w = B // NW
    mesh = plsc.VectorSubcoreMesh(core_axis_name="c", subcore_axis_name="s")
    out_t = jax.ShapeDtypeStruct((B, 2 * EMBED), jnp.float32)

    @functools.partial(
        pl.kernel,
        mesh=mesh,
        out_type=(out_t, out_t),
        scratch_types=[
            pltpu.VMEM((b_per_w,), jnp.int32),
            pltpu.VMEM((b_per_w,), jnp.int32),
            pltpu.VMEM((b_per_w, 2 * EMBED), jnp.float32),
            pltpu.SemaphoreType.DMA,
        ],
    )
    def gather_kernel(ut_hbm, it_hbm, ui_hbm, ii_hbm, ou_hbm, oi_hbm,
                      ui_v, ii_v, rows_v, sem):
        wid = lax.axis_index("s") * NC + lax.axis_index("c")
        base = wid * b_per_w
        pltpu.sync_copy(ui_hbm.at[pl.ds(base, b_per_w)], ui_v)
        pltpu.sync_copy(ii_hbm.at[pl.ds(base, b_per_w)], ii_v)
        pltpu.async_copy(ut_hbm.at[ui_v], rows_v, sem).wait()
        pltpu.sync_copy(rows_v, ou_hbm.at[pl.ds(base, b_per_w)])
        pltpu.async_copy(it_hbm.at[ii_v], rows_v, sem).wait()
        pltpu.sync_copy(rows_v, oi_hbm.at[pl.ds(base, b_per_w)])

    return gather_kernel(user_pairs, item_pairs, user_hids, item_hids)


def _mlp_body(xu_ref, xv_ref, up_ref, vp_ref, w1u_ref, w1v_ref, b1_ref,
              w2_ref, b2_ref, o_ref):
    xu = xu_ref[...]
    xv = xv_ref[...]
    u = jnp.where(up_ref[...] == 1, xu[:, EMBED:], xu[:, :EMBED])
    v = jnp.where(vp_ref[...] == 1, xv[:, EMBED:], xv[:, :EMBED])
    h = (
        lax.dot_general(u, w1u_ref[...], (((1,), (0,)), ((), ())),
                        precision=lax.Precision.HIGHEST,
                        preferred_element_type=jnp.float32)
        + lax.dot_general(v, w1v_ref[...], (((1,), (0,)), ((), ())),
                          precision=lax.Precision.HIGHEST,
                          preferred_element_type=jnp.float32)
        + b1_ref[...]
    )
    h = jnp.maximum(h, 0.0)
    mlp = jnp.sum(h * w2_ref[...].T, axis=1, keepdims=True)
    dot = jnp.sum(u * v, axis=1, keepdims=True)
    o_ref[...] = dot + mlp + b2_ref[...]


def _tc_mlp(xu, xv, u_par, v_par, W1, b1, W2, b2):
    B = xu.shape[0]
    BLK = 2048
    grid = (B // BLK,)
    w1u = W1[:EMBED]
    w1v = W1[EMBED:]
    b1r = b1.reshape(1, HIDDEN)
    b2r = b2.reshape(1, 1)
    return pl.pallas_call(
        _mlp_body,
        grid=grid,
        in_specs=[
            pl.BlockSpec((BLK, 2 * EMBED), lambda i: (i, 0)),
            pl.BlockSpec((BLK, 2 * EMBED), lambda i: (i, 0)),
            pl.BlockSpec((BLK, 1), lambda i: (i, 0)),
            pl.BlockSpec((BLK, 1), lambda i: (i, 0)),
            pl.BlockSpec((EMBED, HIDDEN), lambda i: (0, 0)),
            pl.BlockSpec((EMBED, HIDDEN), lambda i: (0, 0)),
            pl.BlockSpec((1, HIDDEN), lambda i: (0, 0)),
            pl.BlockSpec((HIDDEN, 1), lambda i: (0, 0)),
            pl.BlockSpec((1, 1), lambda i: (0, 0)),
        ],
        out_specs=pl.BlockSpec((BLK, 1), lambda i: (i, 0)),
        out_shape=jax.ShapeDtypeStruct((B, 1), jnp.float32),
    )(xu, xv, u_par, v_par, w1u, w1v, b1r, W2, b2r)


@jax.jit
def kernel(user_ids, item_ids, user_table, item_table, W1, b1, W2, b2):
    n_pairs = user_table.shape[0] // 2
    user_pairs = user_table.reshape(n_pairs, 2 * EMBED)
    item_pairs = item_table.reshape(item_table.shape[0] // 2, 2 * EMBED)
    xu, xv = _sc_gather(user_pairs, item_pairs,
                        user_ids >> 1, item_ids >> 1)
    u_par = (user_ids & 1).reshape(-1, 1)
    v_par = (item_ids & 1).reshape(-1, 1)
    return _tc_mlp(xu, xv, u_par, v_par, W1, b1, W2, b2)


# speedup vs baseline: 1.0789x; 1.0789x over previous
import functools

import jax
import jax.numpy as jnp
from jax import lax
from jax.experimental import pallas as pl
from jax.experimental.pallas import tpu as pltpu
from jax.experimental.pallas import tpu_sc as plsc

EMBED = 64
HIDDEN = 256
NC = 2
NS = 16
NW = NC * NS


def _sc_gather(user_pairs, item_pairs, user_hids, item_hids):
    B = user_hids.shape[0]
    assert B % (8 * NW) == 0
    b_per_w = B // NW
    mesh = plsc.VectorSubcoreMesh(core_axis_name="c", subcore_axis_name="s")
    out_t = jax.ShapeDtypeStruct((B, 2 * EMBED), jnp.float32)

    @functools.partial(
        pl.kernel,
        mesh=mesh,
        out_type=(out_t, out_t),
        scratch_types=[
            pltpu.VMEM((b_per_w,), jnp.int32),
            pltpu.VMEM((b_per_w,), jnp.int32),
            pltpu.VMEM((b_per_w, 2 * EMBED), jnp.float32),
            pltpu.SemaphoreType.DMA,
        ],
    )
    def gather_kernel(ut_hbm, it_hbm, ui_hbm, ii_hbm, ou_hbm, oi_hbm,
                      ui_v, ii_v, rows_v, sem):
        wid = lax.axis_index("s") * NC + lax.axis_index("c")
        base = wid * b_per_w
        pltpu.sync_copy(ui_hbm.at[pl.ds(base, b_per_w)], ui_v)
        pltpu.sync_copy(ii_hbm.at[pl.ds(base, b_per_w)], ii_v)
        pltpu.async_copy(ut_hbm.at[ui_v], rows_v, sem).wait()
        pltpu.sync_copy(rows_v, ou_hbm.at[pl.ds(base, b_per_w)])
        pltpu.async_copy(it_hbm.at[ii_v], rows_v, sem).wait()
        pltpu.sync_copy(rows_v, oi_hbm.at[pl.ds(base, b_per_w)])

    return gather_kernel(user_pairs, item_pairs, user_hids, item_hids)


def _mlp_body(xu_ref, xv_ref, w1u_ref, w1v_ref, b1_ref,
              w2_ref, b2_ref, o_ref):
    u = xu_ref[:, :EMBED]
    v = xv_ref[:, :EMBED]
    h = (
        lax.dot_general(u, w1u_ref[...], (((1,), (0,)), ((), ())),
                        precision=lax.Precision.HIGHEST,
                        preferred_element_type=jnp.float32)
        + lax.dot_general(v, w1v_ref[...], (((1,), (0,)), ((), ())),
                          precision=lax.Precision.HIGHEST,
                          preferred_element_type=jnp.float32)
        + b1_ref[...]
    )
    h = jnp.maximum(h, 0.0)
    mlp = jnp.sum(h * w2_ref[...].T, axis=1, keepdims=True)
    dot = jnp.sum(u * v, axis=1, keepdims=True)
    o_ref[...] = dot + mlp + b2_ref[...]


def _tc_mlp(xu, xv, W1, b1, W2, b2):
    B = xu.shape[0]
    BLK = 2048
    grid = (B // BLK,)
    w1u = W1[:EMBED]
    w1v = W1[EMBED:]
    b1r = b1.reshape(1, HIDDEN)
    b2r = b2.reshape(1, 1)
    return pl.pallas_call(
        _mlp_body,
        grid=grid,
        in_specs=[
            pl.BlockSpec((BLK, 2 * EMBED), lambda i: (i, 0)),
            pl.BlockSpec((BLK, 2 * EMBED), lambda i: (i, 0)),
            pl.BlockSpec((EMBED, HIDDEN), lambda i: (0, 0)),
            pl.BlockSpec((EMBED, HIDDEN), lambda i: (0, 0)),
            pl.BlockSpec((1, HIDDEN), lambda i: (0, 0)),
            pl.BlockSpec((HIDDEN, 1), lambda i: (0, 0)),
            pl.BlockSpec((1, 1), lambda i: (0, 0)),
        ],
        out_specs=pl.BlockSpec((BLK, 1), lambda i: (i, 0)),
        out_shape=jax.ShapeDtypeStruct((B, 1), jnp.float32),
    )(xu, xv, w1u, w1v, b1r, W2, b2r)


@jax.jit
def kernel(user_ids, item_ids, user_table, item_table, W1, b1, W2, b2):
    up = jnp.pad(user_table, ((0, 0), (0, EMBED)))
    ip = jnp.pad(item_table, ((0, 0), (0, EMBED)))
    xu, xv = _sc_gather(up, ip, user_ids, item_ids)
    return _tc_mlp(xu, xv, W1, b1, W2, b2)


# speedup vs baseline: 1.7124x; 1.5873x over previous
import functools

import jax
import jax.numpy as jnp
from jax import lax
from jax.experimental import pallas as pl
from jax.experimental.pallas import tpu as pltpu
from jax.experimental.pallas import tpu_sc as plsc

EMBED = 64
HIDDEN = 256
GRP = 8
NC = 2
NS = 16
NW = NC * NS
CHUNK = 32


def _sc_group_gather(Xu, Xi, ugid, igid):
    B = ugid.shape[0]
    per_w = B // NW
    n_chunks = per_w // CHUNK
    mesh = plsc.VectorSubcoreMesh(core_axis_name="c", subcore_axis_name="s")
    out_t = jax.ShapeDtypeStruct((B, GRP, EMBED), jnp.float32)

    @functools.partial(
        pl.kernel,
        mesh=mesh,
        out_type=(out_t, out_t),
        scratch_types=[
            pltpu.VMEM((per_w,), jnp.int32),
            pltpu.VMEM((per_w,), jnp.int32),
            pltpu.VMEM((CHUNK, GRP, EMBED), jnp.float32),
            pltpu.VMEM((CHUNK, GRP, EMBED), jnp.float32),
            pltpu.SemaphoreType.DMA,
            pltpu.SemaphoreType.DMA,
        ],
    )
    def gather_kernel(xu_hbm, xi_hbm, ug_hbm, ig_hbm, ou_hbm, oi_hbm,
                      ug_v, ig_v, bu_v, bi_v, sem_u, sem_i):
        wid = lax.axis_index("s") * NC + lax.axis_index("c")
        base = wid * per_w
        pltpu.sync_copy(ug_hbm.at[pl.ds(base, per_w)], ug_v)
        pltpu.sync_copy(ig_hbm.at[pl.ds(base, per_w)], ig_v)

        @pl.loop(0, n_chunks)
        def _chunks(c):
            off = c * CHUNK

            @pl.loop(0, CHUNK // 16)
            def _issue(g):
                vu = ug_v[pl.ds(off + g * 16, 16)]
                vi = ig_v[pl.ds(off + g * 16, 16)]
                for k in range(16):
                    pltpu.async_copy(xu_hbm.at[pl.ds(vu[k], 1)],
                                     bu_v.at[pl.ds(g * 16 + k, 1)], sem_u)
                    pltpu.async_copy(xi_hbm.at[pl.ds(vi[k], 1)],
                                     bi_v.at[pl.ds(g * 16 + k, 1)], sem_i)

            @pl.loop(0, CHUNK)
            def _drain(j):
                pltpu.make_async_copy(xu_hbm.at[pl.ds(0, 1)],
                                      bu_v.at[pl.ds(j, 1)], sem_u).wait()
                pltpu.make_async_copy(xi_hbm.at[pl.ds(0, 1)],
                                      bi_v.at[pl.ds(j, 1)], sem_i).wait()

            pltpu.sync_copy(bu_v, ou_hbm.at[pl.ds(base + off, CHUNK)])
            pltpu.sync_copy(bi_v, oi_hbm.at[pl.ds(base + off, CHUNK)])

    return gather_kernel(Xu, Xi, ugid, igid)


def _mlp_body(xu_ref, xv_ref, us_ref, vs_ref, w1u_ref, w1v_ref, b1_ref,
              w2_ref, b2_ref, o_ref):
    us = us_ref[...]
    vs = vs_ref[...]
    u = jnp.zeros_like(xu_ref[:, 0, :])
    v = jnp.zeros_like(u)
    for a in range(GRP):
        u = u + jnp.where(us == a, xu_ref[:, a, :], 0.0)
        v = v + jnp.where(vs == a, xv_ref[:, a, :], 0.0)
    h = (
        lax.dot_general(u, w1u_ref[...], (((1,), (0,)), ((), ())),
                        precision=lax.Precision.HIGHEST,
                        preferred_element_type=jnp.float32)
        + lax.dot_general(v, w1v_ref[...], (((1,), (0,)), ((), ())),
                          precision=lax.Precision.HIGHEST,
                          preferred_element_type=jnp.float32)
        + b1_ref[...]
    )
    h = jnp.maximum(h, 0.0)
    mlp = jnp.sum(h * w2_ref[...].T, axis=1, keepdims=True)
    dot = jnp.sum(u * v, axis=1, keepdims=True)
    o_ref[...] = dot + mlp + b2_ref[...]


def _tc_mlp(xu, xv, u_sel, v_sel, W1, b1, W2, b2):
    B = xu.shape[0]
    BLK = 2048
    grid = (B // BLK,)
    w1u = W1[:EMBED]
    w1v = W1[EMBED:]
    b1r = b1.reshape(1, HIDDEN)
    b2r = b2.reshape(1, 1)
    return pl.pallas_call(
        _mlp_body,
        grid=grid,
        in_specs=[
            pl.BlockSpec((BLK, GRP, EMBED), lambda i: (i, 0, 0)),
            pl.BlockSpec((BLK, GRP, EMBED), lambda i: (i, 0, 0)),
            pl.BlockSpec((BLK, 1), lambda i: (i, 0)),
            pl.BlockSpec((BLK, 1), lambda i: (i, 0)),
            pl.BlockSpec((EMBED, HIDDEN), lambda i: (0, 0)),
            pl.BlockSpec((EMBED, HIDDEN), lambda i: (0, 0)),
            pl.BlockSpec((1, HIDDEN), lambda i: (0, 0)),
            pl.BlockSpec((HIDDEN, 1), lambda i: (0, 0)),
            pl.BlockSpec((1, 1), lambda i: (0, 0)),
        ],
        out_specs=pl.BlockSpec((BLK, 1), lambda i: (i, 0)),
        out_shape=jax.ShapeDtypeStruct((B, 1), jnp.float32),
    )(xu, xv, u_sel, v_sel, w1u, w1v, b1r, W2, b2r)


@jax.jit
def kernel(user_ids, item_ids, user_table, item_table, W1, b1, W2, b2):
    n_grp = user_table.shape[0] // GRP
    Xu = user_table.reshape(n_grp, GRP, EMBED)
    Xi = item_table.reshape(n_grp, GRP, EMBED)
    gu, gi = _sc_group_gather(Xu, Xi, user_ids // GRP, item_ids // GRP)
    u_sel = (user_ids % GRP).reshape(-1, 1)
    v_sel = (item_ids % GRP).reshape(-1, 1)
    return _tc_mlp(gu, gi, u_sel, v_sel, W1, b1, W2, b2)
